# Initial kernel scaffold; baseline (speedup 1.0000x reference)
import jax
import jax.numpy as jnp
from jax.experimental import pallas as pl


def kernel(x_gate, x_experts, noise, Wg, Wn, We, be):
    raise NotImplementedError("write your pallas kernel here")


# speedup vs baseline: 1.7438x; 1.7438x over previous
import functools

import jax
import jax.numpy as jnp
from jax import lax
from jax.experimental import pallas as pl
from jax.experimental.pallas import tpu as pltpu
from jax.experimental.pallas import tpu_sc as plsc

N = 4096
D = 1024
E = 8
K = 2

BT = 512
NT_A = N // BT
TILE = 256
CAP = N * K + E * TILE
NTM = CAP // TILE

NC = 2
NS = 16
NW = NC * NS
TW = N // NW
CS = 64
CC = 32


def _gating_body(x_ref, nt_ref, wg_ref, wn_ref,
                 e0_ref, e1_ref, r0_ref, r1_ref, w0_ref, w1_ref, cnt_ref,
                 carry_ref):
    t = pl.program_id(0)
    x = x_ref[...]
    dn = (((1,), (1,)), ((), ()))
    clean = lax.dot_general(wg_ref[...], x, dn,
                            preferred_element_type=jnp.float32)
    raw = lax.dot_general(wn_ref[...], x, dn,
                          preferred_element_type=jnp.float32)
    softplus = jnp.maximum(raw, 0.0) + jnp.log1p(jnp.exp(-jnp.abs(raw)))
    noisy = clean + nt_ref[...] * softplus

    ii = lax.broadcasted_iota(jnp.int32, (E, BT), 0)
    m1 = jnp.max(noisy, axis=0, keepdims=True)
    i1 = jnp.min(jnp.where(noisy == m1, ii, E), axis=0, keepdims=True)
    sel1 = ii == i1
    masked = jnp.where(sel1, -jnp.inf, noisy)
    m2 = jnp.max(masked, axis=0, keepdims=True)
    i2 = jnp.min(jnp.where(masked == m2, ii, E), axis=0, keepdims=True)
    sel2 = ii == i2
    z = jnp.exp(m2 - m1)
    w0 = 1.0 / (1.0 + z)
    w1 = 1.0 - w0

    @pl.when(t == 0)
    def _():
        carry_ref[...] = jnp.zeros_like(carry_ref)

    onehot = (sel1 | sel2).astype(jnp.int32)
    csum = onehot
    sh = 1
    while sh < BT:
        csum = csum + jnp.concatenate(
            [jnp.zeros((E, sh), jnp.int32), csum[:, :BT - sh]], axis=1)
        sh *= 2
    carry = carry_ref[...]
    rank = carry + csum - onehot
    r0 = jnp.sum(jnp.where(sel1, rank, 0), axis=0, keepdims=True)
    r1 = jnp.sum(jnp.where(sel2, rank, 0), axis=0, keepdims=True)
    carry_ref[...] = carry + csum[:, BT - 1:BT]

    e0_ref[...] = i1.reshape(1, 1, BT)
    e1_ref[...] = i2.reshape(1, 1, BT)
    r0_ref[...] = r0.reshape(1, 1, BT)
    r1_ref[...] = r1.reshape(1, 1, BT)
    w0_ref[...] = w0.reshape(1, 1, BT)
    w1_ref[...] = w1.reshape(1, 1, BT)

    @pl.when(t == NT_A - 1)
    def _():
        cnt_ref[...] = carry + csum[:, BT - 1:BT]


def _gating(x_gate, noise_t, wg, wn):
    small = lambda dt: jax.ShapeDtypeStruct((NT_A, 1, BT), dt)
    sblk = lambda: pl.BlockSpec((1, 1, BT), lambda t: (t, 0, 0))
    return pl.pallas_call(
        _gating_body,
        grid=(NT_A,),
        in_specs=[
            pl.BlockSpec((BT, D), lambda t: (t, 0)),
            pl.BlockSpec((E, BT), lambda t: (0, t)),
            pl.BlockSpec((E, D), lambda t: (0, 0)),
            pl.BlockSpec((E, D), lambda t: (0, 0)),
        ],
        out_specs=[sblk(), sblk(), sblk(), sblk(), sblk(), sblk(),
                   pl.BlockSpec((E, 1), lambda t: (0, 0))],
        out_shape=[small(jnp.int32), small(jnp.int32),
                   small(jnp.int32), small(jnp.int32),
                   small(jnp.float32), small(jnp.float32),
                   jax.ShapeDtypeStruct((E, 1), jnp.int32)],
        scratch_shapes=[pltpu.VMEM((E, 1), jnp.int32)],
    )(x_gate, noise_t, wg, wn)


def _scatter_body(x_hbm, e0_hbm, e1_hbm, r0_hbm, r1_hbm, st_hbm,
                  xg_hbm, i0_hbm, i1_hbm,
                  xv, e0v, e1v, r0v, r1v, i0v, i1v, sv, sem):
    wid = lax.axis_index("s") * NC + lax.axis_index("c")
    pltpu.sync_copy(st_hbm, sv)
    for s in range(TW // CS):
        base = wid * TW + s * CS
        pltpu.sync_copy(x_hbm.at[pl.ds(base, CS)], xv)
        pltpu.sync_copy(e0_hbm.at[pl.ds(base, CS)], e0v)
        pltpu.sync_copy(e1_hbm.at[pl.ds(base, CS)], e1v)
        pltpu.sync_copy(r0_hbm.at[pl.ds(base, CS)], r0v)
        pltpu.sync_copy(r1_hbm.at[pl.ds(base, CS)], r1v)
        for c in range(CS // 16):
            sl = pl.ds(c * 16, 16)
            i0v[sl] = plsc.load_gather(sv, [e0v[sl]]) + r0v[sl]
            i1v[sl] = plsc.load_gather(sv, [e1v[sl]]) + r1v[sl]
        pltpu.async_copy(xv, xg_hbm.at[i0v], sem).wait()
        pltpu.async_copy(xv, xg_hbm.at[i1v], sem).wait()
        pltpu.sync_copy(i0v, i0_hbm.at[pl.ds(base, CS)])
        pltpu.sync_copy(i1v, i1_hbm.at[pl.ds(base, CS)])


@functools.lru_cache(maxsize=None)
def _sc_kernels():
    mesh = plsc.VectorSubcoreMesh(core_axis_name="c", subcore_axis_name="s",
                                  num_cores=NC, num_subcores=NS)
    params = pltpu.CompilerParams(needs_layout_passes=False)
    scatter = pl.kernel(
        _scatter_body,
        compiler_params=params,
        out_type=[jax.ShapeDtypeStruct((CAP, D), jnp.float32),
                  jax.ShapeDtypeStruct((N,), jnp.int32),
                  jax.ShapeDtypeStruct((N,), jnp.int32)],
        mesh=mesh,
        scratch_types=[pltpu.VMEM((CS, D), jnp.float32),
                       pltpu.VMEM((CS,), jnp.int32), pltpu.VMEM((CS,), jnp.int32),
                       pltpu.VMEM((CS,), jnp.int32), pltpu.VMEM((CS,), jnp.int32),
                       pltpu.VMEM((CS,), jnp.int32), pltpu.VMEM((CS,), jnp.int32),
                       pltpu.VMEM((16,), jnp.int32),
                       pltpu.SemaphoreType.DMA],
    )
    combine = pl.kernel(
        _combine_body,
        compiler_params=params,
        out_type=jax.ShapeDtypeStruct((N, D), jnp.float32),
        mesh=mesh,
        scratch_types=[pltpu.VMEM((CC, D), jnp.float32),
                       pltpu.VMEM((CC, D), jnp.float32),
                       pltpu.VMEM((CC, D), jnp.float32),
                       pltpu.VMEM((CC,), jnp.int32), pltpu.VMEM((CC,), jnp.int32),
                       pltpu.VMEM((CC,), jnp.float32), pltpu.VMEM((CC,), jnp.float32),
                       pltpu.SemaphoreType.DMA],
    )
    return scatter, combine


def _expert_mm_body(te_ref, x_ref, w_ref, b_ref, y_ref):
    y_ref[...] = lax.dot_general(
        x_ref[...], w_ref[0], (((1,), (1,)), ((), ())),
        preferred_element_type=jnp.float32) + b_ref[0]


def _expert_mm(tile_expert, xg, we, be):
    grid_spec = pltpu.PrefetchScalarGridSpec(
        num_scalar_prefetch=1,
        grid=(NTM,),
        in_specs=[
            pl.BlockSpec((TILE, D), lambda t, te: (t, 0)),
            pl.BlockSpec((1, D, D), lambda t, te: (te[t], 0, 0)),
            pl.BlockSpec((1, 1, D), lambda t, te: (te[t], 0, 0)),
        ],
        out_specs=pl.BlockSpec((TILE, D), lambda t, te: (t, 0)),
    )
    return pl.pallas_call(
        _expert_mm_body,
        grid_spec=grid_spec,
        out_shape=jax.ShapeDtypeStruct((CAP, D), jnp.float32),
    )(tile_expert, xg, we, be.reshape(E, 1, D))


def _combine_body(y_hbm, i0_hbm, i1_hbm, w0_hbm, w1_hbm, out_hbm,
                  y0v, y1v, ov, i0v, i1v, w0v, w1v, sem):
    wid = lax.axis_index("s") * NC + lax.axis_index("c")
    for s in range(TW // CC):
        base = wid * TW + s * CC
        pltpu.sync_copy(i0_hbm.at[pl.ds(base, CC)], i0v)
        pltpu.sync_copy(i1_hbm.at[pl.ds(base, CC)], i1v)
        pltpu.sync_copy(w0_hbm.at[pl.ds(base, CC)], w0v)
        pltpu.sync_copy(w1_hbm.at[pl.ds(base, CC)], w1v)
        pltpu.async_copy(y_hbm.at[i0v], y0v, sem).wait()
        pltpu.async_copy(y_hbm.at[i1v], y1v, sem).wait()

        def row(r, carry):
            bidx = jnp.zeros((16,), jnp.int32) + r
            wv0 = plsc.load_gather(w0v, [bidx])
            wv1 = plsc.load_gather(w1v, [bidx])
            for j in range(D // 16):
                sl = pl.ds(j * 16, 16)
                ov[r, sl] = wv0 * y0v[r, sl] + wv1 * y1v[r, sl]
            return carry

        lax.fori_loop(0, CC, row, 0)
        pltpu.sync_copy(ov, out_hbm.at[pl.ds(base, CC)])


def kernel(x_gate, x_experts, noise, Wg, Wn, We, be):
    e0, e1, r0, r1, w0, w1, cnt = _gating(x_gate, noise.T, Wg, Wn)
    e0 = e0.reshape(N)
    e1 = e1.reshape(N)
    r0 = r0.reshape(N)
    r1 = r1.reshape(N)
    w0 = w0.reshape(N)
    w1 = w1.reshape(N)

    counts = cnt[:, 0]
    cpad = ((counts + TILE - 1) // TILE) * TILE
    bounds = jnp.cumsum(cpad)
    starts = (bounds - cpad).astype(jnp.int32)
    st16 = jnp.pad(starts, (0, 16 - E))
    tile_base = jnp.arange(NTM, dtype=jnp.int32) * TILE
    tile_expert = jnp.minimum(
        jnp.sum((tile_base[:, None] >= bounds[None, :]).astype(jnp.int32),
                axis=1), E - 1).astype(jnp.int32)

    scatter, combine = _sc_kernels()
    xg, i0, i1 = scatter(x_experts, e0, e1, r0, r1, st16)
    y = _expert_mm(tile_expert, xg, We, be)
    return combine(y, i0, i1, w0, w1)


# speedup vs baseline: 1.9168x; 1.0992x over previous
import functools

import jax
import jax.numpy as jnp
from jax import lax
from jax.experimental import pallas as pl
from jax.experimental.pallas import tpu as pltpu
from jax.experimental.pallas import tpu_sc as plsc

N = 4096
D = 1024
E = 8
K = 2

BT = 512
NT_A = N // BT
TILE = 256
CAP = N * K + E * TILE
NTM = CAP // TILE

NC = 2
NS = 16
NW = NC * NS
TW = N // NW
CSX = 16
CC = 16


def _gating_body(x_ref, nt_ref, wg_ref, wn_ref,
                 e0_ref, e1_ref, r0_ref, r1_ref, w0_ref, w1_ref, cnt_ref,
                 carry_ref):
    t = pl.program_id(0)
    x = x_ref[...]
    dn = (((1,), (1,)), ((), ()))
    clean = lax.dot_general(wg_ref[...], x, dn,
                            preferred_element_type=jnp.float32)
    raw = lax.dot_general(wn_ref[...], x, dn,
                          preferred_element_type=jnp.float32)
    softplus = jnp.maximum(raw, 0.0) + jnp.log1p(jnp.exp(-jnp.abs(raw)))
    noisy = clean + nt_ref[...] * softplus

    ii = lax.broadcasted_iota(jnp.int32, (E, BT), 0)
    m1 = jnp.max(noisy, axis=0, keepdims=True)
    i1 = jnp.min(jnp.where(noisy == m1, ii, E), axis=0, keepdims=True)
    sel1 = ii == i1
    masked = jnp.where(sel1, -jnp.inf, noisy)
    m2 = jnp.max(masked, axis=0, keepdims=True)
    i2 = jnp.min(jnp.where(masked == m2, ii, E), axis=0, keepdims=True)
    sel2 = ii == i2
    z = jnp.exp(m2 - m1)
    w0 = 1.0 / (1.0 + z)
    w1 = 1.0 - w0

    @pl.when(t == 0)
    def _():
        carry_ref[...] = jnp.zeros_like(carry_ref)

    onehot = (sel1 | sel2).astype(jnp.int32)
    csum = onehot
    sh = 1
    while sh < BT:
        csum = csum + jnp.concatenate(
            [jnp.zeros((E, sh), jnp.int32), csum[:, :BT - sh]], axis=1)
        sh *= 2
    carry = carry_ref[...]
    rank = carry + csum - onehot
    r0 = jnp.sum(jnp.where(sel1, rank, 0), axis=0, keepdims=True)
    r1 = jnp.sum(jnp.where(sel2, rank, 0), axis=0, keepdims=True)
    carry_ref[...] = carry + csum[:, BT - 1:BT]

    e0_ref[...] = i1.reshape(1, 1, BT)
    e1_ref[...] = i2.reshape(1, 1, BT)
    r0_ref[...] = r0.reshape(1, 1, BT)
    r1_ref[...] = r1.reshape(1, 1, BT)
    w0_ref[...] = w0.reshape(1, 1, BT)
    w1_ref[...] = w1.reshape(1, 1, BT)

    @pl.when(t == NT_A - 1)
    def _():
        cnt_ref[...] = carry + csum[:, BT - 1:BT]


def _gating(x_gate, noise_t, wg, wn):
    small = lambda dt: jax.ShapeDtypeStruct((NT_A, 1, BT), dt)
    sblk = lambda: pl.BlockSpec((1, 1, BT), lambda t: (t, 0, 0))
    return pl.pallas_call(
        _gating_body,
        grid=(NT_A,),
        in_specs=[
            pl.BlockSpec((BT, D), lambda t: (t, 0)),
            pl.BlockSpec((E, BT), lambda t: (0, t)),
            pl.BlockSpec((E, D), lambda t: (0, 0)),
            pl.BlockSpec((E, D), lambda t: (0, 0)),
        ],
        out_specs=[sblk(), sblk(), sblk(), sblk(), sblk(), sblk(),
                   pl.BlockSpec((E, 1), lambda t: (0, 0))],
        out_shape=[small(jnp.int32), small(jnp.int32),
                   small(jnp.int32), small(jnp.int32),
                   small(jnp.float32), small(jnp.float32),
                   jax.ShapeDtypeStruct((E, 1), jnp.int32)],
        scratch_shapes=[pltpu.VMEM((E, 1), jnp.int32)],
    )(x_gate, noise_t, wg, wn)


def _scatter_body(x_hbm, e0_hbm, e1_hbm, r0_hbm, r1_hbm, st_hbm,
                  xg_hbm, i0_hbm, i1_hbm,
                  xv0, xv1, e0v, e1v, r0v, r1v,
                  i0v, i1v, i0f, i1f, sv,
                  sl0, sl1, ss0, ss1):
    wid = lax.axis_index("s") * NC + lax.axis_index("c")
    wbase = wid * TW
    pltpu.sync_copy(st_hbm, sv)
    pltpu.sync_copy(e0_hbm.at[pl.ds(wbase, TW)], e0v)
    pltpu.sync_copy(e1_hbm.at[pl.ds(wbase, TW)], e1v)
    pltpu.sync_copy(r0_hbm.at[pl.ds(wbase, TW)], r0v)
    pltpu.sync_copy(r1_hbm.at[pl.ds(wbase, TW)], r1v)
    for c in range(TW // 16):
        sl = pl.ds(c * 16, 16)
        v0 = plsc.load_gather(sv, [e0v[sl]]) + r0v[sl]
        v1 = plsc.load_gather(sv, [e1v[sl]]) + r1v[sl]
        i0v[c, :] = v0
        i1v[c, :] = v1
        i0f[sl] = v0
        i1f[sl] = v1
    pltpu.sync_copy(i0f, i0_hbm.at[pl.ds(wbase, TW)])
    pltpu.sync_copy(i1f, i1_hbm.at[pl.ds(wbase, TW)])

    nch = TW // CSX
    bufs = (xv0, xv1)
    lsems = (sl0, sl1)
    ssems = (ss0, ss1)
    loads = [None] * nch
    scats = [None] * nch

    def fire_load(s):
        loads[s] = pltpu.async_copy(
            x_hbm.at[pl.ds(wbase + s * CSX, CSX)], bufs[s % 2], lsems[s % 2])

    def fire_scats(s):
        scats[s] = (
            pltpu.async_copy(bufs[s % 2], xg_hbm.at[i0v.at[s]], ssems[s % 2]),
            pltpu.async_copy(bufs[s % 2], xg_hbm.at[i1v.at[s]], ssems[s % 2]))

    fire_load(0)
    for s in range(nch):
        loads[s].wait()
        fire_scats(s)
        if s + 1 < nch:
            if s >= 1:
                scats[s - 1][0].wait()
                scats[s - 1][1].wait()
            fire_load(s + 1)
    scats[nch - 1][0].wait()
    scats[nch - 1][1].wait()


@functools.lru_cache(maxsize=None)
def _sc_kernels():
    mesh = plsc.VectorSubcoreMesh(core_axis_name="c", subcore_axis_name="s",
                                  num_cores=NC, num_subcores=NS)
    params = pltpu.CompilerParams(needs_layout_passes=False)
    iv = lambda *s: pltpu.VMEM(s, jnp.int32)
    fv = lambda *s: pltpu.VMEM(s, jnp.float32)
    scatter = pl.kernel(
        _scatter_body,
        compiler_params=params,
        out_type=[jax.ShapeDtypeStruct((CAP, D), jnp.float32),
                  jax.ShapeDtypeStruct((N,), jnp.int32),
                  jax.ShapeDtypeStruct((N,), jnp.int32)],
        mesh=mesh,
        scratch_types=[fv(CSX, D), fv(CSX, D),
                       iv(TW), iv(TW), iv(TW), iv(TW),
                       iv(TW // 16, 16), iv(TW // 16, 16), iv(TW), iv(TW),
                       iv(16),
                       pltpu.SemaphoreType.DMA, pltpu.SemaphoreType.DMA,
                       pltpu.SemaphoreType.DMA, pltpu.SemaphoreType.DMA],
    )
    combine = pl.kernel(
        _combine_body,
        compiler_params=params,
        out_type=jax.ShapeDtypeStruct((N, D), jnp.float32),
        mesh=mesh,
        scratch_types=[fv(CC, D), fv(CC, D), fv(CC, D), fv(CC, D),
                       iv(TW // CC, CC), iv(TW // CC, CC),
                       fv(TW // CC, CC), fv(TW // CC, CC),
                       pltpu.SemaphoreType.DMA, pltpu.SemaphoreType.DMA,
                       pltpu.SemaphoreType.DMA, pltpu.SemaphoreType.DMA],
    )
    return scatter, combine


def _expert_mm_body(te_ref, x_ref, w_ref, b_ref, y_ref):
    y_ref[...] = lax.dot_general(
        x_ref[...], w_ref[0], (((1,), (1,)), ((), ())),
        preferred_element_type=jnp.float32) + b_ref[0]


def _expert_mm(tile_expert, xg, we, be):
    grid_spec = pltpu.PrefetchScalarGridSpec(
        num_scalar_prefetch=1,
        grid=(NTM,),
        in_specs=[
            pl.BlockSpec((TILE, D), lambda t, te: (t, 0)),
            pl.BlockSpec((1, D, D), lambda t, te: (te[t], 0, 0)),
            pl.BlockSpec((1, 1, D), lambda t, te: (te[t], 0, 0)),
        ],
        out_specs=pl.BlockSpec((TILE, D), lambda t, te: (t, 0)),
    )
    return pl.pallas_call(
        _expert_mm_body,
        grid_spec=grid_spec,
        out_shape=jax.ShapeDtypeStruct((CAP, D), jnp.float32),
    )(tile_expert, xg, we, be.reshape(E, 1, D))


def _combine_body(y_hbm, i0_hbm, i1_hbm, w0_hbm, w1_hbm, out_hbm,
                  y0a, y0b, y1a, y1b, i0v, i1v, w0v, w1v,
                  sg0, sg1, st0, st1):
    wid = lax.axis_index("s") * NC + lax.axis_index("c")
    wbase = wid * TW
    pltpu.sync_copy(i0_hbm.at[wid], i0v)
    pltpu.sync_copy(i1_hbm.at[wid], i1v)
    pltpu.sync_copy(w0_hbm.at[wid], w0v)
    pltpu.sync_copy(w1_hbm.at[wid], w1v)

    nch = TW // CC
    b0 = (y0a, y0b)
    b1 = (y1a, y1b)
    gsems = (sg0, sg1)
    tsems = (st0, st1)
    gath = [None] * nch
    stor = [None] * nch

    def fire_gathers(s):
        gath[s] = (
            pltpu.async_copy(y_hbm.at[i0v.at[s]], b0[s % 2], gsems[s % 2]),
            pltpu.async_copy(y_hbm.at[i1v.at[s]], b1[s % 2], gsems[s % 2]))

    fire_gathers(0)
    for s in range(nch):
        if s + 1 < nch:
            if s >= 1:
                stor[s - 1].wait()
            fire_gathers(s + 1)
        gath[s][0].wait()
        gath[s][1].wait()
        y0 = b0[s % 2]
        y1 = b1[s % 2]
        srow = jnp.full((16,), s, jnp.int32)

        def row(r, carry):
            bidx = jnp.zeros((16,), jnp.int32) + r
            wv0 = plsc.load_gather(w0v, [srow, bidx])
            wv1 = plsc.load_gather(w1v, [srow, bidx])
            for j in range(D // 16):
                sl = pl.ds(j * 16, 16)
                y0[r, sl] = wv0 * y0[r, sl] + wv1 * y1[r, sl]
            return carry

        lax.fori_loop(0, CC, row, 0)
        stor[s] = pltpu.async_copy(
            y0, out_hbm.at[pl.ds(wbase + s * CC, CC)], tsems[s % 2])
    stor[nch - 2].wait()
    stor[nch - 1].wait()


def kernel(x_gate, x_experts, noise, Wg, Wn, We, be):
    e0, e1, r0, r1, w0, w1, cnt = _gating(x_gate, noise.T, Wg, Wn)
    e0 = e0.reshape(N)
    e1 = e1.reshape(N)
    r0 = r0.reshape(N)
    r1 = r1.reshape(N)
    w0 = w0.reshape(N)
    w1 = w1.reshape(N)

    counts = cnt[:, 0]
    cpad = ((counts + TILE - 1) // TILE) * TILE
    bounds = jnp.cumsum(cpad)
    starts = (bounds - cpad).astype(jnp.int32)
    st16 = jnp.pad(starts, (0, 16 - E))
    tile_base = jnp.arange(NTM, dtype=jnp.int32) * TILE
    tile_expert = jnp.minimum(
        jnp.sum((tile_base[:, None] >= bounds[None, :]).astype(jnp.int32),
                axis=1), E - 1).astype(jnp.int32)

    scatter, combine = _sc_kernels()
    xg, i0, i1 = scatter(x_experts, e0, e1, r0, r1, st16)
    y = _expert_mm(tile_expert, xg, We, be)
    i0r = i0.reshape(NW, TW // CC, CC)
    i1r = i1.reshape(NW, TW // CC, CC)
    w0r = w0.reshape(NW, TW // CC, CC)
    w1r = w1.reshape(NW, TW // CC, CC)
    return combine(y, i0r, i1r, w0r, w1r)


# speedup vs baseline: 1.9515x; 1.0181x over previous
import functools

import jax
import jax.numpy as jnp
from jax import lax
from jax.experimental import pallas as pl
from jax.experimental.pallas import tpu as pltpu
from jax.experimental.pallas import tpu_sc as plsc

N = 4096
NSHIFT = 12
D = 1024
E = 8
K = 2

BT = 512
NT_A = N // BT
TILE = 256
CAP = N * K + E * TILE
NTM = CAP // TILE

NC = 2
NS = 16
NW = NC * NS
TW = N // NW
CSX = 16
CC = 16


def _gating_body(x_ref, nt_ref, wg_ref, wn_ref,
                 e0_ref, e1_ref, w0_ref, w1_ref, cnt_ref,
                 carry_ref):
    t = pl.program_id(0)
    x = x_ref[...]
    dn = (((1,), (1,)), ((), ()))
    clean = lax.dot_general(wg_ref[...], x, dn,
                            preferred_element_type=jnp.float32)
    raw = lax.dot_general(wn_ref[...], x, dn,
                          preferred_element_type=jnp.float32)
    softplus = jnp.maximum(raw, 0.0) + jnp.log1p(jnp.exp(-jnp.abs(raw)))
    noisy = clean + nt_ref[...] * softplus

    ii = lax.broadcasted_iota(jnp.int32, (E, BT), 0)
    m1 = jnp.max(noisy, axis=0, keepdims=True)
    i1 = jnp.min(jnp.where(noisy == m1, ii, E), axis=0, keepdims=True)
    sel1 = ii == i1
    masked = jnp.where(sel1, -jnp.inf, noisy)
    m2 = jnp.max(masked, axis=0, keepdims=True)
    i2 = jnp.min(jnp.where(masked == m2, ii, E), axis=0, keepdims=True)
    sel2 = ii == i2
    z = jnp.exp(m2 - m1)
    w0 = 1.0 / (1.0 + z)
    w1 = 1.0 - w0

    @pl.when(t == 0)
    def _():
        carry_ref[...] = jnp.zeros_like(carry_ref)

    onehot = (sel1 | sel2).astype(jnp.int32)
    csum = onehot
    sh = 1
    while sh < BT:
        csum = csum + jnp.concatenate(
            [jnp.zeros((E, sh), jnp.int32), csum[:, :BT - sh]], axis=1)
        sh *= 2
    carry = carry_ref[...]
    rank = carry + csum - onehot
    r0 = jnp.sum(jnp.where(sel1, rank, 0), axis=0, keepdims=True)
    r1 = jnp.sum(jnp.where(sel2, rank, 0), axis=0, keepdims=True)
    carry_ref[...] = carry + csum[:, BT - 1:BT]

    e0_ref[...] = (i1 * N + r0).reshape(BT)
    e1_ref[...] = (i2 * N + r1).reshape(BT)
    w0_ref[...] = w0.reshape(BT)
    w1_ref[...] = w1.reshape(BT)

    @pl.when(t == NT_A - 1)
    def _():
        cnt_ref[...] = carry + csum[:, BT - 1:BT]


def _gating(x_gate, noise_t, wg, wn):
    flat = lambda dt: jax.ShapeDtypeStruct((N,), dt)
    fblk = lambda: pl.BlockSpec((BT,), lambda t: (t,))
    return pl.pallas_call(
        _gating_body,
        grid=(NT_A,),
        in_specs=[
            pl.BlockSpec((BT, D), lambda t: (t, 0)),
            pl.BlockSpec((E, BT), lambda t: (0, t)),
            pl.BlockSpec((E, D), lambda t: (0, 0)),
            pl.BlockSpec((E, D), lambda t: (0, 0)),
        ],
        out_specs=[fblk(), fblk(), fblk(), fblk(),
                   pl.BlockSpec((E, 1), lambda t: (0, 0))],
        out_shape=[flat(jnp.int32), flat(jnp.int32),
                   flat(jnp.float32), flat(jnp.float32),
                   jax.ShapeDtypeStruct((E, 1), jnp.int32)],
        scratch_shapes=[pltpu.VMEM((E, 1), jnp.int32)],
    )(x_gate, noise_t, wg, wn)


def _scatter_body(x_hbm, p0_hbm, p1_hbm, st_hbm,
                  xg_hbm,
                  xv0, xv1, p0v, p1v, i0v, i1v, sv,
                  sl0, sl1, ss0, ss1):
    wid = lax.axis_index("s") * NC + lax.axis_index("c")
    wbase = wid * TW
    pltpu.sync_copy(st_hbm, sv)
    pltpu.sync_copy(p0_hbm.at[pl.ds(wbase, TW)], p0v)
    pltpu.sync_copy(p1_hbm.at[pl.ds(wbase, TW)], p1v)
    for c in range(TW // 16):
        sl = pl.ds(c * 16, 16)
        pk0 = p0v[sl]
        pk1 = p1v[sl]
        i0v[c, :] = plsc.load_gather(sv, [pk0 >> NSHIFT]) + (pk0 & (N - 1))
        i1v[c, :] = plsc.load_gather(sv, [pk1 >> NSHIFT]) + (pk1 & (N - 1))

    nch = TW // CSX
    bufs = (xv0, xv1)
    lsems = (sl0, sl1)
    ssems = (ss0, ss1)
    loads = [None] * nch
    scats = [None] * nch

    def fire_load(s):
        loads[s] = pltpu.async_copy(
            x_hbm.at[pl.ds(wbase + s * CSX, CSX)], bufs[s % 2], lsems[s % 2])

    def fire_scats(s):
        scats[s] = (
            pltpu.async_copy(bufs[s % 2], xg_hbm.at[i0v.at[s]], ssems[s % 2]),
            pltpu.async_copy(bufs[s % 2], xg_hbm.at[i1v.at[s]], ssems[s % 2]))

    fire_load(0)
    for s in range(nch):
        loads[s].wait()
        fire_scats(s)
        if s + 1 < nch:
            if s >= 1:
                scats[s - 1][0].wait()
                scats[s - 1][1].wait()
            fire_load(s + 1)
    scats[nch - 1][0].wait()
    scats[nch - 1][1].wait()


@functools.lru_cache(maxsize=None)
def _sc_kernels():
    mesh = plsc.VectorSubcoreMesh(core_axis_name="c", subcore_axis_name="s",
                                  num_cores=NC, num_subcores=NS)
    params = pltpu.CompilerParams(needs_layout_passes=False)
    iv = lambda *s: pltpu.VMEM(s, jnp.int32)
    fv = lambda *s: pltpu.VMEM(s, jnp.float32)
    scatter = pl.kernel(
        _scatter_body,
        compiler_params=params,
        out_type=jax.ShapeDtypeStruct((CAP, D), jnp.float32),
        mesh=mesh,
        scratch_types=[fv(CSX, D), fv(CSX, D),
                       iv(TW), iv(TW),
                       iv(TW // 16, 16), iv(TW // 16, 16),
                       iv(16),
                       pltpu.SemaphoreType.DMA, pltpu.SemaphoreType.DMA,
                       pltpu.SemaphoreType.DMA, pltpu.SemaphoreType.DMA],
    )
    combine = pl.kernel(
        _combine_body,
        compiler_params=params,
        out_type=jax.ShapeDtypeStruct((N, D), jnp.float32),
        mesh=mesh,
        scratch_types=[fv(CC, D), fv(CC, D), fv(CC, D), fv(CC, D),
                       iv(TW), iv(TW),
                       iv(TW // CC, CC), iv(TW // CC, CC),
                       fv(TW), fv(TW), iv(16),
                       pltpu.SemaphoreType.DMA, pltpu.SemaphoreType.DMA,
                       pltpu.SemaphoreType.DMA, pltpu.SemaphoreType.DMA],
    )
    return scatter, combine


def _expert_mm_body(te_ref, x_ref, w_ref, b_ref, y_ref):
    y_ref[...] = lax.dot_general(
        x_ref[...], w_ref[0], (((1,), (1,)), ((), ())),
        preferred_element_type=jnp.float32) + b_ref[0]


def _expert_mm(tile_expert, xg, we, be):
    grid_spec = pltpu.PrefetchScalarGridSpec(
        num_scalar_prefetch=1,
        grid=(NTM,),
        in_specs=[
            pl.BlockSpec((TILE, D), lambda t, te: (t, 0)),
            pl.BlockSpec((1, D, D), lambda t, te: (te[t], 0, 0)),
            pl.BlockSpec((1, 1, D), lambda t, te: (te[t], 0, 0)),
        ],
        out_specs=pl.BlockSpec((TILE, D), lambda t, te: (t, 0)),
    )
    return pl.pallas_call(
        _expert_mm_body,
        grid_spec=grid_spec,
        out_shape=jax.ShapeDtypeStruct((CAP, D), jnp.float32),
    )(tile_expert, xg, we, be.reshape(E, 1, D))


def _combine_body(y_hbm, p0_hbm, p1_hbm, w0_hbm, w1_hbm, st_hbm, out_hbm,
                  y0a, y0b, y1a, y1b, p0v, p1v, i0v, i1v, w0v, w1v, sv,
                  sg0, sg1, st0, st1):
    wid = lax.axis_index("s") * NC + lax.axis_index("c")
    wbase = wid * TW
    pltpu.sync_copy(st_hbm, sv)
    pltpu.sync_copy(p0_hbm.at[pl.ds(wbase, TW)], p0v)
    pltpu.sync_copy(p1_hbm.at[pl.ds(wbase, TW)], p1v)
    pltpu.sync_copy(w0_hbm.at[pl.ds(wbase, TW)], w0v)
    pltpu.sync_copy(w1_hbm.at[pl.ds(wbase, TW)], w1v)
    for c in range(TW // 16):
        sl = pl.ds(c * 16, 16)
        pk0 = p0v[sl]
        pk1 = p1v[sl]
        i0v[c, :] = plsc.load_gather(sv, [pk0 >> NSHIFT]) + (pk0 & (N - 1))
        i1v[c, :] = plsc.load_gather(sv, [pk1 >> NSHIFT]) + (pk1 & (N - 1))

    nch = TW // CC
    b0 = (y0a, y0b)
    b1 = (y1a, y1b)
    gsems = (sg0, sg1)
    tsems = (st0, st1)
    gath = [None] * nch
    stor = [None] * nch

    def fire_gathers(s):
        gath[s] = (
            pltpu.async_copy(y_hbm.at[i0v.at[s]], b0[s % 2], gsems[s % 2]),
            pltpu.async_copy(y_hbm.at[i1v.at[s]], b1[s % 2], gsems[s % 2]))

    fire_gathers(0)
    for s in range(nch):
        if s + 1 < nch:
            if s >= 1:
                stor[s - 1].wait()
            fire_gathers(s + 1)
        gath[s][0].wait()
        gath[s][1].wait()
        y0 = b0[s % 2]
        y1 = b1[s % 2]

        def row(r, carry):
            bidx = jnp.full((16,), s * CC, jnp.int32) + r
            wv0 = plsc.load_gather(w0v, [bidx])
            wv1 = plsc.load_gather(w1v, [bidx])
            for j in range(D // 16):
                sl = pl.ds(j * 16, 16)
                y0[r, sl] = wv0 * y0[r, sl] + wv1 * y1[r, sl]
            return carry

        lax.fori_loop(0, CC, row, 0)
        stor[s] = pltpu.async_copy(
            y0, out_hbm.at[pl.ds(wbase + s * CC, CC)], tsems[s % 2])
    stor[nch - 2].wait()
    stor[nch - 1].wait()


def kernel(x_gate, x_experts, noise, Wg, Wn, We, be):
    p0, p1, w0, w1, cnt = _gating(x_gate, noise.T, Wg, Wn)

    counts = cnt[:, 0]
    cpad = ((counts + TILE - 1) // TILE) * TILE
    bounds = jnp.cumsum(cpad)
    starts = (bounds - cpad).astype(jnp.int32)
    st16 = jnp.pad(starts, (0, 16 - E))
    tile_base = jnp.arange(NTM, dtype=jnp.int32) * TILE
    tile_expert = jnp.minimum(
        jnp.sum((tile_base[:, None] >= bounds[None, :]).astype(jnp.int32),
                axis=1), E - 1).astype(jnp.int32)

    scatter, combine = _sc_kernels()
    xg = scatter(x_experts, p0, p1, st16)
    y = _expert_mm(tile_expert, xg, We, be)
    return combine(y, p0, p1, w0, w1, st16)


# speedup vs baseline: 1.9645x; 1.0066x over previous
import functools

import jax
import jax.numpy as jnp
from jax import lax
from jax.experimental import pallas as pl
from jax.experimental.pallas import tpu as pltpu
from jax.experimental.pallas import tpu_sc as plsc

N = 4096
NSHIFT = 12
D = 1024
E = 8
K = 2

BT = 512
NT_A = N // BT
TILE = 256
CAP = N * K + E * TILE
NTM = CAP // TILE

NC = 2
NS = 16
NW = NC * NS
TW = N // NW
CSX = 16
CC = 16


def _gating_body(x_ref, nt_ref, wg_ref, wn_ref,
                 e0_ref, e1_ref, w0_ref, w1_ref, cnt_ref,
                 carry_ref):
    t = pl.program_id(0)
    x = x_ref[...]
    dn = (((1,), (1,)), ((), ()))
    clean = lax.dot_general(wg_ref[...], x, dn,
                            preferred_element_type=jnp.float32)
    raw = lax.dot_general(wn_ref[...], x, dn,
                          preferred_element_type=jnp.float32)
    softplus = jnp.maximum(raw, 0.0) + jnp.log1p(jnp.exp(-jnp.abs(raw)))
    noisy = clean + nt_ref[...] * softplus

    ii = lax.broadcasted_iota(jnp.int32, (E, BT), 0)
    m1 = jnp.max(noisy, axis=0, keepdims=True)
    i1 = jnp.min(jnp.where(noisy == m1, ii, E), axis=0, keepdims=True)
    sel1 = ii == i1
    masked = jnp.where(sel1, -jnp.inf, noisy)
    m2 = jnp.max(masked, axis=0, keepdims=True)
    i2 = jnp.min(jnp.where(masked == m2, ii, E), axis=0, keepdims=True)
    sel2 = ii == i2
    z = jnp.exp(m2 - m1)
    w0 = 1.0 / (1.0 + z)
    w1 = 1.0 - w0

    @pl.when(t == 0)
    def _():
        carry_ref[...] = jnp.zeros_like(carry_ref)

    onehot = (sel1 | sel2).astype(jnp.int32)
    csum = onehot
    sh = 1
    while sh < BT:
        csum = csum + jnp.concatenate(
            [jnp.zeros((E, sh), jnp.int32), csum[:, :BT - sh]], axis=1)
        sh *= 2
    carry = carry_ref[...]
    rank = carry + csum - onehot
    r0 = jnp.sum(jnp.where(sel1, rank, 0), axis=0, keepdims=True)
    r1 = jnp.sum(jnp.where(sel2, rank, 0), axis=0, keepdims=True)
    carry_ref[...] = carry + csum[:, BT - 1:BT]

    e0_ref[...] = (i1 * N + r0).reshape(BT)
    e1_ref[...] = (i2 * N + r1).reshape(BT)
    w0_ref[...] = w0.reshape(BT)
    w1_ref[...] = w1.reshape(BT)

    @pl.when(t == NT_A - 1)
    def _():
        cnt_ref[...] = carry + csum[:, BT - 1:BT]


def _gating(x_gate, noise_t, wg, wn):
    flat = lambda dt: jax.ShapeDtypeStruct((N,), dt)
    fblk = lambda: pl.BlockSpec((BT,), lambda t: (t,))
    return pl.pallas_call(
        _gating_body,
        grid=(NT_A,),
        in_specs=[
            pl.BlockSpec((BT, D), lambda t: (t, 0)),
            pl.BlockSpec((E, BT), lambda t: (0, t)),
            pl.BlockSpec((E, D), lambda t: (0, 0)),
            pl.BlockSpec((E, D), lambda t: (0, 0)),
        ],
        out_specs=[fblk(), fblk(), fblk(), fblk(),
                   pl.BlockSpec((E, 1), lambda t: (0, 0))],
        out_shape=[flat(jnp.int32), flat(jnp.int32),
                   flat(jnp.float32), flat(jnp.float32),
                   jax.ShapeDtypeStruct((E, 1), jnp.int32)],
        scratch_shapes=[pltpu.VMEM((E, 1), jnp.int32)],
    )(x_gate, noise_t, wg, wn)


def _scatter_body(x_hbm, p0_hbm, p1_hbm, st_hbm,
                  xg_hbm,
                  xv0, xv1, p0v, p1v, i0v, i1v, sv,
                  sl0, sl1, ss0, ss1):
    wid = lax.axis_index("s") * NC + lax.axis_index("c")
    wbase = wid * TW
    pltpu.sync_copy(st_hbm, sv)
    pltpu.sync_copy(p0_hbm.at[pl.ds(wbase, TW)], p0v)
    pltpu.sync_copy(p1_hbm.at[pl.ds(wbase, TW)], p1v)
    for c in range(TW // 16):
        sl = pl.ds(c * 16, 16)
        pk0 = p0v[sl]
        pk1 = p1v[sl]
        i0v[c, :] = plsc.load_gather(sv, [pk0 >> NSHIFT]) + (pk0 & (N - 1))
        i1v[c, :] = plsc.load_gather(sv, [pk1 >> NSHIFT]) + (pk1 & (N - 1))

    nch = TW // CSX
    bufs = (xv0, xv1)
    lsems = (sl0, sl1)
    ssems = (ss0, ss1)
    loads = [None] * nch
    scats = [None] * nch

    def fire_load(s):
        loads[s] = pltpu.async_copy(
            x_hbm.at[pl.ds(wbase + s * CSX, CSX)], bufs[s % 2], lsems[s % 2])

    def fire_scats(s):
        scats[s] = (
            pltpu.async_copy(bufs[s % 2], xg_hbm.at[i0v.at[s]], ssems[s % 2]),
            pltpu.async_copy(bufs[s % 2], xg_hbm.at[i1v.at[s]], ssems[s % 2]))

    fire_load(0)
    for s in range(nch):
        loads[s].wait()
        fire_scats(s)
        if s + 1 < nch:
            if s >= 1:
                scats[s - 1][0].wait()
                scats[s - 1][1].wait()
            fire_load(s + 1)
    scats[nch - 1][0].wait()
    scats[nch - 1][1].wait()


@functools.lru_cache(maxsize=None)
def _sc_kernels():
    mesh = plsc.VectorSubcoreMesh(core_axis_name="c", subcore_axis_name="s",
                                  num_cores=NC, num_subcores=NS)
    params = pltpu.CompilerParams(needs_layout_passes=False)
    iv = lambda *s: pltpu.VMEM(s, jnp.int32)
    fv = lambda *s: pltpu.VMEM(s, jnp.float32)
    scatter = pl.kernel(
        _scatter_body,
        compiler_params=params,
        out_type=jax.ShapeDtypeStruct((CAP, D), jnp.float32),
        mesh=mesh,
        scratch_types=[fv(CSX, D), fv(CSX, D),
                       iv(TW), iv(TW),
                       iv(TW // 16, 16), iv(TW // 16, 16),
                       iv(16),
                       pltpu.SemaphoreType.DMA, pltpu.SemaphoreType.DMA,
                       pltpu.SemaphoreType.DMA, pltpu.SemaphoreType.DMA],
    )
    combine = pl.kernel(
        _combine_body,
        compiler_params=params,
        out_type=jax.ShapeDtypeStruct((N, D), jnp.float32),
        mesh=mesh,
        scratch_types=[fv(CC, D), fv(CC, D), fv(CC, D), fv(CC, D),
                       iv(TW), iv(TW),
                       iv(TW // CC, CC), iv(TW // CC, CC),
                       fv(TW), fv(TW), iv(16),
                       pltpu.SemaphoreType.DMA, pltpu.SemaphoreType.DMA,
                       pltpu.SemaphoreType.DMA, pltpu.SemaphoreType.DMA],
    )
    return scatter, combine


def _expert_mm_body(te_ref, x_ref, w_ref, b_ref, y_ref):
    e = te_ref[pl.program_id(0)]
    w = w_ref[pl.ds(e, 1)][0]
    y_ref[...] = lax.dot_general(
        x_ref[...], w, (((1,), (1,)), ((), ())),
        preferred_element_type=jnp.float32) + b_ref[pl.ds(e, 1)][0]


def _expert_mm(tile_expert, xg, we, be):
    grid_spec = pltpu.PrefetchScalarGridSpec(
        num_scalar_prefetch=1,
        grid=(NTM,),
        in_specs=[
            pl.BlockSpec((TILE, D), lambda t, te: (t, 0)),
            pl.BlockSpec((E, D, D), lambda t, te: (0, 0, 0)),
            pl.BlockSpec((E, 1, D), lambda t, te: (0, 0, 0)),
        ],
        out_specs=pl.BlockSpec((TILE, D), lambda t, te: (t, 0)),
    )
    return pl.pallas_call(
        _expert_mm_body,
        grid_spec=grid_spec,
        out_shape=jax.ShapeDtypeStruct((CAP, D), jnp.float32),
    )(tile_expert, xg, we, be.reshape(E, 1, D))


def _combine_body(y_hbm, p0_hbm, p1_hbm, w0_hbm, w1_hbm, st_hbm, out_hbm,
                  y0a, y0b, y1a, y1b, p0v, p1v, i0v, i1v, w0v, w1v, sv,
                  sg0, sg1, st0, st1):
    wid = lax.axis_index("s") * NC + lax.axis_index("c")
    wbase = wid * TW
    pltpu.sync_copy(st_hbm, sv)
    pltpu.sync_copy(p0_hbm.at[pl.ds(wbase, TW)], p0v)
    pltpu.sync_copy(p1_hbm.at[pl.ds(wbase, TW)], p1v)
    pltpu.sync_copy(w0_hbm.at[pl.ds(wbase, TW)], w0v)
    pltpu.sync_copy(w1_hbm.at[pl.ds(wbase, TW)], w1v)
    for c in range(TW // 16):
        sl = pl.ds(c * 16, 16)
        pk0 = p0v[sl]
        pk1 = p1v[sl]
        i0v[c, :] = plsc.load_gather(sv, [pk0 >> NSHIFT]) + (pk0 & (N - 1))
        i1v[c, :] = plsc.load_gather(sv, [pk1 >> NSHIFT]) + (pk1 & (N - 1))

    nch = TW // CC
    b0 = (y0a, y0b)
    b1 = (y1a, y1b)
    gsems = (sg0, sg1)
    tsems = (st0, st1)
    gath = [None] * nch
    stor = [None] * nch

    def fire_gathers(s):
        gath[s] = (
            pltpu.async_copy(y_hbm.at[i0v.at[s]], b0[s % 2], gsems[s % 2]),
            pltpu.async_copy(y_hbm.at[i1v.at[s]], b1[s % 2], gsems[s % 2]))

    fire_gathers(0)
    for s in range(nch):
        if s + 1 < nch:
            if s >= 1:
                stor[s - 1].wait()
            fire_gathers(s + 1)
        gath[s][0].wait()
        gath[s][1].wait()
        y0 = b0[s % 2]
        y1 = b1[s % 2]

        def row(r, carry):
            bidx = jnp.full((16,), s * CC, jnp.int32) + r
            wv0 = plsc.load_gather(w0v, [bidx])
            wv1 = plsc.load_gather(w1v, [bidx])
            for j in range(D // 16):
                sl = pl.ds(j * 16, 16)
                y0[r, sl] = wv0 * y0[r, sl] + wv1 * y1[r, sl]
            return carry

        lax.fori_loop(0, CC, row, 0)
        stor[s] = pltpu.async_copy(
            y0, out_hbm.at[pl.ds(wbase + s * CC, CC)], tsems[s % 2])
    stor[nch - 2].wait()
    stor[nch - 1].wait()


def kernel(x_gate, x_experts, noise, Wg, Wn, We, be):
    p0, p1, w0, w1, cnt = _gating(x_gate, noise.T, Wg, Wn)

    counts = cnt[:, 0]
    cpad = ((counts + TILE - 1) // TILE) * TILE
    bounds = jnp.cumsum(cpad)
    starts = (bounds - cpad).astype(jnp.int32)
    st16 = jnp.pad(starts, (0, 16 - E))
    tile_base = jnp.arange(NTM, dtype=jnp.int32) * TILE
    tile_expert = jnp.minimum(
        jnp.sum((tile_base[:, None] >= bounds[None, :]).astype(jnp.int32),
                axis=1), E - 1).astype(jnp.int32)

    scatter, combine = _sc_kernels()
    xg = scatter(x_experts, p0, p1, st16)
    y = _expert_mm(tile_expert, xg, We, be)
    return combine(y, p0, p1, w0, w1, st16)


# speedup vs baseline: 2.0607x; 1.0490x over previous
import functools

import jax
import jax.numpy as jnp
from jax import lax
from jax.experimental import pallas as pl
from jax.experimental.pallas import tpu as pltpu
from jax.experimental.pallas import tpu_sc as plsc

N = 4096
NSHIFT = 12
D = 1024
D2 = D // 2
E = 8
K = 2

BT = 512
NT_A = N // BT
TILE = 256
CAP = N * K + E * TILE
NTM = CAP // TILE

NC = 2
NS = 16
NW = NC * NS
TW = N // NW
CSX = 16
CC = 16


def _gating_body(x_ref, nt_ref, wg_ref, wn_ref, xe_ref,
                 e0_ref, e1_ref, w0_ref, w1_ref, xbf_ref, cnt_ref,
                 carry_ref):
    t = pl.program_id(0)
    xb = lax.bitcast_convert_type(xe_ref[...], jnp.int32) + 0x8000
    lo = lax.shift_right_logical(xb[:, :D2], 16)
    hi = xb[:, D2:] & jnp.int32(-65536)
    xbf_ref[...] = lo | hi
    x = x_ref[...]
    dn = (((1,), (1,)), ((), ()))
    clean = lax.dot_general(wg_ref[...], x, dn,
                            preferred_element_type=jnp.float32)
    raw = lax.dot_general(wn_ref[...], x, dn,
                          preferred_element_type=jnp.float32)
    softplus = jnp.maximum(raw, 0.0) + jnp.log1p(jnp.exp(-jnp.abs(raw)))
    noisy = clean + nt_ref[...] * softplus

    ii = lax.broadcasted_iota(jnp.int32, (E, BT), 0)
    m1 = jnp.max(noisy, axis=0, keepdims=True)
    i1 = jnp.min(jnp.where(noisy == m1, ii, E), axis=0, keepdims=True)
    sel1 = ii == i1
    masked = jnp.where(sel1, -jnp.inf, noisy)
    m2 = jnp.max(masked, axis=0, keepdims=True)
    i2 = jnp.min(jnp.where(masked == m2, ii, E), axis=0, keepdims=True)
    sel2 = ii == i2
    z = jnp.exp(m2 - m1)
    w0 = 1.0 / (1.0 + z)
    w1 = 1.0 - w0

    @pl.when(t == 0)
    def _():
        carry_ref[...] = jnp.zeros_like(carry_ref)

    onehot = (sel1 | sel2).astype(jnp.int32)
    csum = onehot
    sh = 1
    while sh < BT:
        csum = csum + jnp.concatenate(
            [jnp.zeros((E, sh), jnp.int32), csum[:, :BT - sh]], axis=1)
        sh *= 2
    carry = carry_ref[...]
    rank = carry + csum - onehot
    r0 = jnp.sum(jnp.where(sel1, rank, 0), axis=0, keepdims=True)
    r1 = jnp.sum(jnp.where(sel2, rank, 0), axis=0, keepdims=True)
    carry_ref[...] = carry + csum[:, BT - 1:BT]

    e0_ref[...] = (i1 * N + r0).reshape(BT)
    e1_ref[...] = (i2 * N + r1).reshape(BT)
    w0_ref[...] = w0.reshape(BT)
    w1_ref[...] = w1.reshape(BT)

    @pl.when(t == NT_A - 1)
    def _():
        cnt_ref[...] = carry + csum[:, BT - 1:BT]


def _gating(x_gate, noise_t, wg, wn, x_experts):
    flat = lambda dt: jax.ShapeDtypeStruct((N,), dt)
    fblk = lambda: pl.BlockSpec((BT,), lambda t: (t,))
    return pl.pallas_call(
        _gating_body,
        grid=(NT_A,),
        in_specs=[
            pl.BlockSpec((BT, D), lambda t: (t, 0)),
            pl.BlockSpec((E, BT), lambda t: (0, t)),
            pl.BlockSpec((E, D), lambda t: (0, 0)),
            pl.BlockSpec((E, D), lambda t: (0, 0)),
            pl.BlockSpec((BT, D), lambda t: (t, 0)),
        ],
        out_specs=[fblk(), fblk(), fblk(), fblk(),
                   pl.BlockSpec((BT, D2), lambda t: (t, 0)),
                   pl.BlockSpec((E, 1), lambda t: (0, 0))],
        out_shape=[flat(jnp.int32), flat(jnp.int32),
                   flat(jnp.float32), flat(jnp.float32),
                   jax.ShapeDtypeStruct((N, D2), jnp.int32),
                   jax.ShapeDtypeStruct((E, 1), jnp.int32)],
        scratch_shapes=[pltpu.VMEM((E, 1), jnp.int32)],
    )(x_gate, noise_t, wg, wn, x_experts)


def _scatter_body(x_hbm, p0_hbm, p1_hbm, st_hbm,
                  xg_hbm,
                  xv0, xv1, p0v, p1v, i0v, i1v, sv,
                  sl0, sl1, ss0, ss1):
    wid = lax.axis_index("s") * NC + lax.axis_index("c")
    wbase = wid * TW
    pltpu.sync_copy(st_hbm, sv)
    pltpu.sync_copy(p0_hbm.at[pl.ds(wbase, TW)], p0v)
    pltpu.sync_copy(p1_hbm.at[pl.ds(wbase, TW)], p1v)
    for c in range(TW // 16):
        sl = pl.ds(c * 16, 16)
        pk0 = p0v[sl]
        pk1 = p1v[sl]
        i0v[c, :] = plsc.load_gather(sv, [pk0 >> NSHIFT]) + (pk0 & (N - 1))
        i1v[c, :] = plsc.load_gather(sv, [pk1 >> NSHIFT]) + (pk1 & (N - 1))

    nch = TW // CSX
    bufs = (xv0, xv1)
    lsems = (sl0, sl1)
    ssems = (ss0, ss1)
    loads = [None] * nch
    scats = [None] * nch

    def fire_load(s):
        loads[s] = pltpu.async_copy(
            x_hbm.at[pl.ds(wbase + s * CSX, CSX)], bufs[s % 2], lsems[s % 2])

    def fire_scats(s):
        scats[s] = (
            pltpu.async_copy(bufs[s % 2], xg_hbm.at[i0v.at[s]], ssems[s % 2]),
            pltpu.async_copy(bufs[s % 2], xg_hbm.at[i1v.at[s]], ssems[s % 2]))

    fire_load(0)
    for s in range(nch):
        loads[s].wait()
        fire_scats(s)
        if s + 1 < nch:
            if s >= 1:
                scats[s - 1][0].wait()
                scats[s - 1][1].wait()
            fire_load(s + 1)
    scats[nch - 1][0].wait()
    scats[nch - 1][1].wait()


@functools.lru_cache(maxsize=None)
def _sc_kernels():
    mesh = plsc.VectorSubcoreMesh(core_axis_name="c", subcore_axis_name="s",
                                  num_cores=NC, num_subcores=NS)
    params = pltpu.CompilerParams(needs_layout_passes=False)
    iv = lambda *s: pltpu.VMEM(s, jnp.int32)
    fv = lambda *s: pltpu.VMEM(s, jnp.float32)
    scatter = pl.kernel(
        _scatter_body,
        compiler_params=params,
        out_type=jax.ShapeDtypeStruct((CAP, D2), jnp.int32),
        mesh=mesh,
        scratch_types=[iv(CSX, D2), iv(CSX, D2),
                       iv(TW), iv(TW),
                       iv(TW // 16, 16), iv(TW // 16, 16),
                       iv(16),
                       pltpu.SemaphoreType.DMA, pltpu.SemaphoreType.DMA,
                       pltpu.SemaphoreType.DMA, pltpu.SemaphoreType.DMA],
    )
    combine = pl.kernel(
        _combine_body,
        compiler_params=params,
        out_type=jax.ShapeDtypeStruct((N, D), jnp.float32),
        mesh=mesh,
        scratch_types=[fv(CC, D), fv(CC, D), fv(CC, D), fv(CC, D),
                       iv(TW), iv(TW),
                       iv(TW // CC, CC), iv(TW // CC, CC),
                       fv(TW), fv(TW), iv(16),
                       pltpu.SemaphoreType.DMA, pltpu.SemaphoreType.DMA,
                       pltpu.SemaphoreType.DMA, pltpu.SemaphoreType.DMA],
    )
    return scatter, combine


def _expert_mm_body(te_ref, x_ref, w_ref, b_ref, y_ref):
    e = te_ref[pl.program_id(0)]
    w = w_ref[pl.ds(e, 1)][0]
    xi = x_ref[...]
    xlo = lax.bitcast_convert_type(xi << 16, jnp.float32)
    xhi = lax.bitcast_convert_type(xi & jnp.int32(-65536), jnp.float32)
    dn = (((1,), (1,)), ((), ()))
    y_ref[...] = (
        lax.dot_general(xlo, w[:, :D2], dn, preferred_element_type=jnp.float32)
        + lax.dot_general(xhi, w[:, D2:], dn, preferred_element_type=jnp.float32)
        + b_ref[pl.ds(e, 1)][0])


def _expert_mm(tile_expert, xg, we, be):
    grid_spec = pltpu.PrefetchScalarGridSpec(
        num_scalar_prefetch=1,
        grid=(NTM,),
        in_specs=[
            pl.BlockSpec((TILE, D2), lambda t, te: (t, 0)),
            pl.BlockSpec((E, D, D), lambda t, te: (0, 0, 0)),
            pl.BlockSpec((E, 1, D), lambda t, te: (0, 0, 0)),
        ],
        out_specs=pl.BlockSpec((TILE, D), lambda t, te: (t, 0)),
    )
    return pl.pallas_call(
        _expert_mm_body,
        grid_spec=grid_spec,
        out_shape=jax.ShapeDtypeStruct((CAP, D), jnp.float32),
    )(tile_expert, xg, we, be.reshape(E, 1, D))


def _combine_body(y_hbm, p0_hbm, p1_hbm, w0_hbm, w1_hbm, st_hbm, out_hbm,
                  y0a, y0b, y1a, y1b, p0v, p1v, i0v, i1v, w0v, w1v, sv,
                  sg0, sg1, st0, st1):
    wid = lax.axis_index("s") * NC + lax.axis_index("c")
    wbase = wid * TW
    pltpu.sync_copy(st_hbm, sv)
    pltpu.sync_copy(p0_hbm.at[pl.ds(wbase, TW)], p0v)
    pltpu.sync_copy(p1_hbm.at[pl.ds(wbase, TW)], p1v)
    pltpu.sync_copy(w0_hbm.at[pl.ds(wbase, TW)], w0v)
    pltpu.sync_copy(w1_hbm.at[pl.ds(wbase, TW)], w1v)
    for c in range(TW // 16):
        sl = pl.ds(c * 16, 16)
        pk0 = p0v[sl]
        pk1 = p1v[sl]
        i0v[c, :] = plsc.load_gather(sv, [pk0 >> NSHIFT]) + (pk0 & (N - 1))
        i1v[c, :] = plsc.load_gather(sv, [pk1 >> NSHIFT]) + (pk1 & (N - 1))

    nch = TW // CC
    b0 = (y0a, y0b)
    b1 = (y1a, y1b)
    gsems = (sg0, sg1)
    tsems = (st0, st1)
    gath = [None] * nch
    stor = [None] * nch

    def fire_gathers(s):
        gath[s] = (
            pltpu.async_copy(y_hbm.at[i0v.at[s]], b0[s % 2], gsems[s % 2]),
            pltpu.async_copy(y_hbm.at[i1v.at[s]], b1[s % 2], gsems[s % 2]))

    fire_gathers(0)
    for s in range(nch):
        if s + 1 < nch:
            if s >= 1:
                stor[s - 1].wait()
            fire_gathers(s + 1)
        gath[s][0].wait()
        gath[s][1].wait()
        y0 = b0[s % 2]
        y1 = b1[s % 2]

        def row(r, carry):
            bidx = jnp.full((16,), s * CC, jnp.int32) + r
            wv0 = plsc.load_gather(w0v, [bidx])
            wv1 = plsc.load_gather(w1v, [bidx])
            for j in range(D // 16):
                sl = pl.ds(j * 16, 16)
                y0[r, sl] = wv0 * y0[r, sl] + wv1 * y1[r, sl]
            return carry

        lax.fori_loop(0, CC, row, 0)
        stor[s] = pltpu.async_copy(
            y0, out_hbm.at[pl.ds(wbase + s * CC, CC)], tsems[s % 2])
    stor[nch - 2].wait()
    stor[nch - 1].wait()


def kernel(x_gate, x_experts, noise, Wg, Wn, We, be):
    p0, p1, w0, w1, xbf, cnt = _gating(x_gate, noise.T, Wg, Wn, x_experts)

    counts = cnt[:, 0]
    cpad = ((counts + TILE - 1) // TILE) * TILE
    bounds = jnp.cumsum(cpad)
    starts = (bounds - cpad).astype(jnp.int32)
    st16 = jnp.pad(starts, (0, 16 - E))
    tile_base = jnp.arange(NTM, dtype=jnp.int32) * TILE
    tile_expert = jnp.minimum(
        jnp.sum((tile_base[:, None] >= bounds[None, :]).astype(jnp.int32),
                axis=1), E - 1).astype(jnp.int32)

    scatter, combine = _sc_kernels()
    xg = scatter(xbf, p0, p1, st16)
    y = _expert_mm(tile_expert, xg, We, be)
    return combine(y, p0, p1, w0, w1, st16)
